# Initial kernel scaffold; baseline (speedup 1.0000x reference)
import jax
import jax.numpy as jnp
from jax.experimental import pallas as pl


def kernel(x, Wq, bq, Wk, bk, Wv, bv, Wo, bo, g1, be1, g2, be2, Wf1, bf1, Wf2, bf2, res_w):
    raise NotImplementedError("write your pallas kernel here")


# speedup vs baseline: 17.4076x; 17.4076x over previous
import functools
import math

import jax
import jax.numpy as jnp
from jax.experimental import pallas as pl
from jax.experimental.pallas import tpu as pltpu

NEG = -1e30


def _proj_body(x_ref, wq_ref, bq_ref, wk_ref, bk_ref, wv_ref, bv_ref,
               g1_ref, be1_ref, nq_ref, nk_ref, vf_ref):
    x = x_ref[...]
    f32 = jnp.float32
    dot = functools.partial(jax.lax.dot_general,
                            dimension_numbers=(((1,), (0,)), ((), ())),
                            preferred_element_type=f32)
    q = dot(x, wq_ref[...]) + bq_ref[...]
    k = dot(x, wk_ref[...]) + bk_ref[...]
    qn = jnp.sqrt(jnp.sum(q * q, axis=1, keepdims=True))
    kn = jnp.sqrt(jnp.sum(k * k, axis=1, keepdims=True))
    nq_ref[...] = q / jnp.maximum(qn, 1e-12)
    nk_ref[...] = k / jnp.maximum(kn, 1e-12)
    m = jnp.mean(x, axis=1, keepdims=True)
    xc = x - m
    v = jnp.mean(xc * xc, axis=1, keepdims=True)
    xln = xc * jax.lax.rsqrt(v + 1e-5) * g1_ref[...] + be1_ref[...]
    vf_ref[...] = dot(xln, wv_ref[...]) + bv_ref[...]


def _attn_body(nq_ref, nk_ref, vf_ref, x_ref, wo_ref, bo_ref, rw_ref,
               h1_ref, *, nk_count, heads):
    nq = nq_ref[0]
    nk = nk_ref[0]
    vf = vf_ref[0]
    dqk = nq.shape[1]
    d = vf.shape[1]
    hq = dqk // heads
    hv = d // heads
    dotT = functools.partial(jax.lax.dot_general,
                             dimension_numbers=(((1,), (1,)), ((), ())),
                             preferred_element_type=jnp.float32)
    dot = functools.partial(jax.lax.dot_general,
                            dimension_numbers=(((1,), (0,)), ((), ())),
                            preferred_element_type=jnp.float32)
    sim = dotT(nq, nk)
    work = sim
    for _ in range(nk_count - 1):
        mx = jnp.max(work, axis=1, keepdims=True)
        work = jnp.where(work == mx, NEG, work)
    thresh = jnp.max(work, axis=1, keepdims=True)
    mask = sim >= thresh
    scale = jnp.float32(1.0 / math.sqrt(hq))
    outs = []
    for h in range(heads):
        qh = nq[:, h * hq:(h + 1) * hq]
        kh = nk[:, h * hq:(h + 1) * hq]
        lh = dotT(qh, kh) * scale
        lh = jnp.where(mask, lh, NEG)
        mh = jnp.max(lh, axis=1, keepdims=True)
        e = jnp.exp(lh - mh)
        e = jnp.where(mask, e, 0.0)
        att = e / jnp.sum(e, axis=1, keepdims=True)
        outs.append(dot(att, vf[:, h * hv:(h + 1) * hv]))
    sa = jnp.concatenate(outs, axis=1)
    sa = dot(sa, wo_ref[...]) + bo_ref[...]
    h1_ref[0] = x_ref[0] + sa * rw_ref[...]


def _ffn_body(h1_ref, g2_ref, be2_ref, wf1_ref, bf1_ref, wf2_ref, bf2_ref,
              rw_ref, out_ref):
    h1 = h1_ref[...]
    dot = functools.partial(jax.lax.dot_general,
                            dimension_numbers=(((1,), (0,)), ((), ())),
                            preferred_element_type=jnp.float32)
    m = jnp.mean(h1, axis=1, keepdims=True)
    hc = h1 - m
    v = jnp.mean(hc * hc, axis=1, keepdims=True)
    hln = hc * jax.lax.rsqrt(v + 1e-5) * g2_ref[...] + be2_ref[...]
    a = dot(hln, wf1_ref[...]) + bf1_ref[...]
    g = 0.5 * a * (1.0 + jax.lax.erf(a * jnp.float32(1.0 / math.sqrt(2.0))))
    ff = dot(g, wf2_ref[...]) + bf2_ref[...]
    out_ref[...] = h1 + ff * rw_ref[...]


def kernel(x, Wq, bq, Wk, bk, Wv, bv, Wo, bo, g1, be1, g2, be2, Wf1, bf1,
           Wf2, bf2, res_w):
    B, N, D = x.shape
    DQK = Wq.shape[1]
    DFF = Wf1.shape[1]
    H = 8
    NKN = 16
    BN = B * N
    f32 = jnp.float32

    x2 = x.reshape(BN, D)
    row = lambda a: a.reshape(1, -1)
    rw = res_w.reshape(1, 1)

    RA = 512
    nq2, nk2, vf2 = pl.pallas_call(
        _proj_body,
        grid=(BN // RA,),
        in_specs=[
            pl.BlockSpec((RA, D), lambda i: (i, 0)),
            pl.BlockSpec((D, DQK), lambda i: (0, 0)),
            pl.BlockSpec((1, DQK), lambda i: (0, 0)),
            pl.BlockSpec((D, DQK), lambda i: (0, 0)),
            pl.BlockSpec((1, DQK), lambda i: (0, 0)),
            pl.BlockSpec((D, D), lambda i: (0, 0)),
            pl.BlockSpec((1, D), lambda i: (0, 0)),
            pl.BlockSpec((1, D), lambda i: (0, 0)),
            pl.BlockSpec((1, D), lambda i: (0, 0)),
        ],
        out_specs=[
            pl.BlockSpec((RA, DQK), lambda i: (i, 0)),
            pl.BlockSpec((RA, DQK), lambda i: (i, 0)),
            pl.BlockSpec((RA, D), lambda i: (i, 0)),
        ],
        out_shape=[
            jax.ShapeDtypeStruct((BN, DQK), f32),
            jax.ShapeDtypeStruct((BN, DQK), f32),
            jax.ShapeDtypeStruct((BN, D), f32),
        ],
    )(x2, Wq, row(bq), Wk, row(bk), Wv, row(bv), row(g1), row(be1))

    nq3 = nq2.reshape(B, N, DQK)
    nk3 = nk2.reshape(B, N, DQK)
    vf3 = vf2.reshape(B, N, D)

    RC = 256
    h1 = pl.pallas_call(
        functools.partial(_attn_body, nk_count=NKN, heads=H),
        grid=(B, N // RC),
        in_specs=[
            pl.BlockSpec((1, RC, DQK), lambda b, i: (b, i, 0)),
            pl.BlockSpec((1, N, DQK), lambda b, i: (b, 0, 0)),
            pl.BlockSpec((1, N, D), lambda b, i: (b, 0, 0)),
            pl.BlockSpec((1, RC, D), lambda b, i: (b, i, 0)),
            pl.BlockSpec((D, D), lambda b, i: (0, 0)),
            pl.BlockSpec((1, D), lambda b, i: (0, 0)),
            pl.BlockSpec((1, 1), lambda b, i: (0, 0)),
        ],
        out_specs=pl.BlockSpec((1, RC, D), lambda b, i: (b, i, 0)),
        out_shape=jax.ShapeDtypeStruct((B, N, D), f32),
    )(nq3, nk3, vf3, x, Wo, row(bo), rw)

    h12 = h1.reshape(BN, D)
    RD = 512
    out = pl.pallas_call(
        _ffn_body,
        grid=(BN // RD,),
        in_specs=[
            pl.BlockSpec((RD, D), lambda i: (i, 0)),
            pl.BlockSpec((1, D), lambda i: (0, 0)),
            pl.BlockSpec((1, D), lambda i: (0, 0)),
            pl.BlockSpec((D, DFF), lambda i: (0, 0)),
            pl.BlockSpec((1, DFF), lambda i: (0, 0)),
            pl.BlockSpec((DFF, D), lambda i: (0, 0)),
            pl.BlockSpec((1, D), lambda i: (0, 0)),
            pl.BlockSpec((1, 1), lambda i: (0, 0)),
        ],
        out_specs=pl.BlockSpec((RD, D), lambda i: (i, 0)),
        out_shape=jax.ShapeDtypeStruct((BN, D), f32),
    )(h12, row(g2), row(be2), Wf1, row(bf1), Wf2, row(bf2), rw)

    return out.reshape(B, N, D)
